# baseline (speedup 1.0000x reference)
import functools

import jax
import jax.numpy as jnp
from jax import lax
from jax.experimental import pallas as pl
from jax.experimental.pallas import tpu as pltpu
from jax.experimental.pallas import tpu_sc as plsc

N, E, D = 10000, 320000, 128
NC, NS = 2, 16
NW = NC * NS
EPW = E // NW
K = 80
C = 125
NP = 10240
RPT = NP // NS


def _sc_partials(x, src, dst, zeros):
    mesh = plsc.VectorSubcoreMesh(core_axis_name="c", subcore_axis_name="s")

    @functools.partial(
        pl.kernel,
        out_type=jax.ShapeDtypeStruct((NC, NP, D), jnp.float32),
        mesh=mesh,
        scratch_types=[
            pltpu.VMEM((2, K), jnp.int32),
            pltpu.VMEM((C, K), jnp.int32),
            pltpu.VMEM((3, K, D), jnp.float32),
            pltpu.VMEM_SHARED((NP, D), jnp.float32),
            pltpu.SemaphoreType.DMA,
            pltpu.SemaphoreType.DMA,
            pltpu.SemaphoreType.DMA,
        ],
    )
    def sc_kernel(x_hbm, src_hbm, dst_hbm, zeros_hbm, out_hbm,
                  src_v, dst_v, rows_v, agg_sh, sem_g, sem_i, sem_s):
        cid = lax.axis_index("c")
        sid = lax.axis_index("s")
        wid = sid * NC + cid
        pltpu.sync_copy(zeros_hbm.at[pl.ds(sid * RPT, RPT)],
                        agg_sh.at[pl.ds(sid * RPT, RPT)])
        pltpu.sync_copy(dst_hbm.at[wid], dst_v)
        plsc.subcore_barrier()

        ibase = wid * EPW
        pltpu.sync_copy(src_hbm.at[pl.ds(ibase, K)], src_v.at[0])
        pltpu.async_copy(x_hbm.at[src_v.at[0]], rows_v.at[0], sem_g)
        pltpu.async_copy(src_hbm.at[pl.ds(ibase + K, K)], src_v.at[1], sem_i)

        def body(g, carry):
            @pl.when(g >= 2)
            def _free_buf():
                pltpu.make_async_copy(
                    rows_v.at[(g + 1) % 3],
                    agg_sh.at[dst_v.at[jnp.maximum(g - 2, 0)]], sem_s).wait()

            pltpu.make_async_copy(
                src_hbm.at[pl.ds(ibase + jnp.minimum(g + 1, C - 1) * K, K)],
                src_v.at[(g + 1) % 2], sem_i).wait()
            pltpu.async_copy(x_hbm.at[src_v.at[(g + 1) % 2]],
                             rows_v.at[(g + 1) % 3], sem_g)
            pltpu.make_async_copy(x_hbm.at[src_v.at[g % 2]],
                                  rows_v.at[g % 3], sem_g).wait()
            pltpu.async_copy(
                src_hbm.at[pl.ds(ibase + jnp.minimum(g + 2, C - 1) * K, K)],
                src_v.at[g % 2], sem_i)
            pltpu.async_copy(rows_v.at[g % 3], agg_sh.at[dst_v.at[g]],
                             sem_s, add=True)
            return carry

        lax.fori_loop(0, C, body, 0)
        pltpu.make_async_copy(x_hbm.at[src_v.at[C % 2]],
                              rows_v.at[C % 3], sem_g).wait()
        pltpu.make_async_copy(src_hbm.at[pl.ds(ibase + (C - 1) * K, K)],
                              src_v.at[(C - 1) % 2], sem_i).wait()

        def drain(g, carry):
            pltpu.make_async_copy(rows_v.at[g % 3],
                                  agg_sh.at[dst_v.at[g]], sem_s).wait()
            return carry

        lax.fori_loop(C - 2, C, drain, 0)
        plsc.subcore_barrier()
        pltpu.sync_copy(agg_sh.at[pl.ds(sid * RPT, RPT)],
                        out_hbm.at[cid].at[pl.ds(sid * RPT, RPT)])

    return sc_kernel(x, src, dst, zeros)


BM = 1000


def _tc_combine(x, p, w1, w2, b2d):
    def body(x_ref, p_ref, w1_ref, w2_ref, b_ref, o_ref):
        agg = p_ref[0] + p_ref[1]
        dn = (((1,), (1,)), ((), ()))
        o_ref[...] = (
            lax.dot_general(x_ref[...], w1_ref[...], dn,
                            preferred_element_type=jnp.float32)
            + lax.dot_general(agg, w2_ref[...], dn,
                              preferred_element_type=jnp.float32)
            + b_ref[...]
        )

    return pl.pallas_call(
        body,
        grid=(N // BM,),
        in_specs=[
            pl.BlockSpec((BM, D), lambda i: (i, 0)),
            pl.BlockSpec((NC, BM, D), lambda i: (0, i, 0)),
            pl.BlockSpec((D, D), lambda i: (0, 0)),
            pl.BlockSpec((D, D), lambda i: (0, 0)),
            pl.BlockSpec((1, D), lambda i: (0, 0)),
        ],
        out_specs=pl.BlockSpec((BM, D), lambda i: (i, 0)),
        out_shape=jax.ShapeDtypeStruct((N, D), jnp.float32),
    )(x, p, w1, w2, b2d)


def kernel(x, edge_index, W, b):
    src = edge_index[0]
    dst = edge_index[1].reshape(NW, C, K)
    zeros = jnp.zeros((NP, D), jnp.float32)
    p = _sc_partials(x, src, dst, zeros)
    w1 = W[:, :D]
    w2 = W[:, D:]
    return _tc_combine(x, p, w1, w2, b.reshape(1, D))


# speedup vs baseline: 1.0027x; 1.0027x over previous
import functools

import jax
import jax.numpy as jnp
from jax import lax
from jax.experimental import pallas as pl
from jax.experimental.pallas import tpu as pltpu
from jax.experimental.pallas import tpu_sc as plsc

N, E, D = 10000, 320000, 128
NC, NS = 2, 16
NW = NC * NS
EPW = E // NW
K = 80
C = 125
NP = 10240
RPT = NP // NS


def _sc_partials(x, src, dst, zeros):
    mesh = plsc.VectorSubcoreMesh(core_axis_name="c", subcore_axis_name="s")

    @functools.partial(
        pl.kernel,
        out_type=jax.ShapeDtypeStruct((NC, NP, D), jnp.float32),
        mesh=mesh,
        scratch_types=[
            pltpu.VMEM((2, K), jnp.int32),
            pltpu.VMEM((C, K), jnp.int32),
            pltpu.VMEM((3, K, D), jnp.float32),
            pltpu.VMEM_SHARED((NP, D), jnp.float32),
            pltpu.SemaphoreType.DMA,
            pltpu.SemaphoreType.DMA,
            pltpu.SemaphoreType.DMA,
        ],
    )
    def sc_kernel(x_hbm, src_hbm, dst_hbm, zeros_hbm, out_hbm,
                  src_v, dst_v, rows_v, agg_sh, sem_g, sem_i, sem_s):
        cid = lax.axis_index("c")
        sid = lax.axis_index("s")
        wid = sid * NC + cid
        pltpu.sync_copy(zeros_hbm.at[pl.ds(sid * RPT, RPT)],
                        agg_sh.at[pl.ds(sid * RPT, RPT)])
        pltpu.sync_copy(dst_hbm.at[wid], dst_v)
        plsc.subcore_barrier()

        ibase = wid * EPW
        pltpu.sync_copy(src_hbm.at[pl.ds(ibase, K)], src_v.at[0])
        pltpu.async_copy(x_hbm.at[src_v.at[0]], rows_v.at[0], sem_g)
        pltpu.async_copy(src_hbm.at[pl.ds(ibase + K, K)], src_v.at[1], sem_i)

        def body(g, carry):
            @pl.when(g >= 2)
            def _free_buf():
                pltpu.make_async_copy(
                    rows_v.at[(g + 1) % 3],
                    agg_sh.at[dst_v.at[jnp.maximum(g - 2, 0)]], sem_s).wait()

            pltpu.make_async_copy(
                src_hbm.at[pl.ds(ibase + jnp.minimum(g + 1, C - 1) * K, K)],
                src_v.at[(g + 1) % 2], sem_i).wait()
            pltpu.async_copy(x_hbm.at[src_v.at[(g + 1) % 2]],
                             rows_v.at[(g + 1) % 3], sem_g)
            pltpu.make_async_copy(x_hbm.at[src_v.at[g % 2]],
                                  rows_v.at[g % 3], sem_g).wait()
            pltpu.async_copy(
                src_hbm.at[pl.ds(ibase + jnp.minimum(g + 2, C - 1) * K, K)],
                src_v.at[g % 2], sem_i)
            pltpu.async_copy(rows_v.at[g % 3], agg_sh.at[dst_v.at[g]],
                             sem_s, add=True)
            return carry

        lax.fori_loop(0, C, body, 0)
        pltpu.make_async_copy(x_hbm.at[src_v.at[C % 2]],
                              rows_v.at[C % 3], sem_g).wait()
        pltpu.make_async_copy(src_hbm.at[pl.ds(ibase + (C - 1) * K, K)],
                              src_v.at[(C - 1) % 2], sem_i).wait()

        def drain(g, carry):
            pltpu.make_async_copy(rows_v.at[g % 3],
                                  agg_sh.at[dst_v.at[g]], sem_s).wait()
            return carry

        lax.fori_loop(C - 2, C, drain, 0)
        plsc.subcore_barrier()
        pltpu.sync_copy(agg_sh.at[pl.ds(sid * RPT, RPT)],
                        out_hbm.at[cid].at[pl.ds(sid * RPT, RPT)])

    return sc_kernel(x, src, dst, zeros)


BM = 1000


def _tc_self(x, w1, b2d):
    def body(x_ref, w1_ref, b_ref, o_ref):
        dn = (((1,), (1,)), ((), ()))
        o_ref[...] = lax.dot_general(
            x_ref[...], w1_ref[...], dn,
            preferred_element_type=jnp.float32) + b_ref[...]

    return pl.pallas_call(
        body,
        grid=(N // BM,),
        in_specs=[
            pl.BlockSpec((BM, D), lambda i: (i, 0)),
            pl.BlockSpec((D, D), lambda i: (0, 0)),
            pl.BlockSpec((1, D), lambda i: (0, 0)),
        ],
        out_specs=pl.BlockSpec((BM, D), lambda i: (i, 0)),
        out_shape=jax.ShapeDtypeStruct((N, D), jnp.float32),
    )(x, w1, b2d)


def _tc_combine(y, p, w2):
    def body(y_ref, p_ref, w2_ref, o_ref):
        agg = p_ref[0] + p_ref[1]
        dn = (((1,), (1,)), ((), ()))
        o_ref[...] = y_ref[...] + lax.dot_general(
            agg, w2_ref[...], dn, preferred_element_type=jnp.float32)

    return pl.pallas_call(
        body,
        grid=(N // BM,),
        in_specs=[
            pl.BlockSpec((BM, D), lambda i: (i, 0)),
            pl.BlockSpec((NC, BM, D), lambda i: (0, i, 0)),
            pl.BlockSpec((D, D), lambda i: (0, 0)),
        ],
        out_specs=pl.BlockSpec((BM, D), lambda i: (i, 0)),
        out_shape=jax.ShapeDtypeStruct((N, D), jnp.float32),
    )(y, p, w2)


def kernel(x, edge_index, W, b):
    src = edge_index[0]
    dst = edge_index[1].reshape(NW, C, K)
    zeros = jnp.zeros((NP, D), jnp.float32)
    p = _sc_partials(x, src, dst, zeros)
    w1 = W[:, :D]
    w2 = W[:, D:]
    y = _tc_self(x, w1, b.reshape(1, D))
    return _tc_combine(y, p, w2)


# speedup vs baseline: 1.0333x; 1.0305x over previous
import functools

import jax
import jax.numpy as jnp
from jax import lax
from jax.experimental import pallas as pl
from jax.experimental.pallas import tpu as pltpu
from jax.experimental.pallas import tpu_sc as plsc

N, E, D = 10000, 320000, 128
NC, NS = 2, 16
NW = NC * NS
EPW = E // NW
K = 80
C = 125
NP = 10240
RPT = NP // NS


def _sc_partials(x, src, dst):
    mesh = plsc.VectorSubcoreMesh(core_axis_name="c", subcore_axis_name="s")

    @functools.partial(
        pl.kernel,
        out_type=jax.ShapeDtypeStruct((NC, NP, D), jnp.float32),
        mesh=mesh,
        scratch_types=[
            pltpu.VMEM((2, K), jnp.int32),
            pltpu.VMEM((C, K), jnp.int32),
            pltpu.VMEM((3, K, D), jnp.float32),
            pltpu.VMEM_SHARED((NP, D), jnp.float32),
            pltpu.SemaphoreType.DMA,
            pltpu.SemaphoreType.DMA,
            pltpu.SemaphoreType.DMA,
        ],
    )
    def sc_kernel(x_hbm, src_hbm, dst_hbm, out_hbm,
                  src_v, dst_v, rows_v, agg_sh, sem_g, sem_i, sem_s):
        cid = lax.axis_index("c")
        sid = lax.axis_index("s")
        wid = sid * NC + cid
        zv = jnp.zeros((16,), jnp.float32)

        def zero_row(i, carry):
            for j in range(8):
                rows_v[0, i, pl.ds(16 * j, 16)] = zv
            return carry

        lax.fori_loop(0, K, zero_row, 0)

        def zero_slice(r, carry):
            pltpu.sync_copy(rows_v.at[0],
                            agg_sh.at[pl.ds(sid * RPT + r * K, K)])
            return carry

        lax.fori_loop(0, RPT // K, zero_slice, 0)
        pltpu.sync_copy(dst_hbm.at[wid], dst_v)
        plsc.subcore_barrier()

        ibase = wid * EPW
        pltpu.sync_copy(src_hbm.at[pl.ds(ibase, K)], src_v.at[0])
        pltpu.async_copy(x_hbm.at[src_v.at[0]], rows_v.at[0], sem_g)
        pltpu.async_copy(src_hbm.at[pl.ds(ibase + K, K)], src_v.at[1], sem_i)

        def body(g, carry):
            @pl.when(g >= 2)
            def _free_buf():
                pltpu.make_async_copy(
                    rows_v.at[(g + 1) % 3],
                    agg_sh.at[dst_v.at[jnp.maximum(g - 2, 0)]], sem_s).wait()

            pltpu.make_async_copy(
                src_hbm.at[pl.ds(ibase + jnp.minimum(g + 1, C - 1) * K, K)],
                src_v.at[(g + 1) % 2], sem_i).wait()
            pltpu.async_copy(x_hbm.at[src_v.at[(g + 1) % 2]],
                             rows_v.at[(g + 1) % 3], sem_g)
            pltpu.make_async_copy(x_hbm.at[src_v.at[g % 2]],
                                  rows_v.at[g % 3], sem_g).wait()
            pltpu.async_copy(
                src_hbm.at[pl.ds(ibase + jnp.minimum(g + 2, C - 1) * K, K)],
                src_v.at[g % 2], sem_i)
            pltpu.async_copy(rows_v.at[g % 3], agg_sh.at[dst_v.at[g]],
                             sem_s, add=True)
            return carry

        lax.fori_loop(0, C, body, 0)
        pltpu.make_async_copy(x_hbm.at[src_v.at[C % 2]],
                              rows_v.at[C % 3], sem_g).wait()
        pltpu.make_async_copy(src_hbm.at[pl.ds(ibase + (C - 1) * K, K)],
                              src_v.at[(C - 1) % 2], sem_i).wait()

        def drain(g, carry):
            pltpu.make_async_copy(rows_v.at[g % 3],
                                  agg_sh.at[dst_v.at[g]], sem_s).wait()
            return carry

        lax.fori_loop(C - 2, C, drain, 0)
        plsc.subcore_barrier()
        pltpu.sync_copy(agg_sh.at[pl.ds(sid * RPT, RPT)],
                        out_hbm.at[cid].at[pl.ds(sid * RPT, RPT)])

    return sc_kernel(x, src, dst)


BM = 1000


def _tc_combine(x, p, w1, w2, b2d):
    def body(x_ref, p_ref, w1_ref, w2_ref, b_ref, o_ref):
        agg = p_ref[0] + p_ref[1]
        dn = (((1,), (1,)), ((), ()))
        o_ref[...] = (
            lax.dot_general(x_ref[...], w1_ref[...], dn,
                            preferred_element_type=jnp.float32)
            + lax.dot_general(agg, w2_ref[...], dn,
                              preferred_element_type=jnp.float32)
            + b_ref[...]
        )

    return pl.pallas_call(
        body,
        grid=(N // BM,),
        in_specs=[
            pl.BlockSpec((BM, D), lambda i: (i, 0)),
            pl.BlockSpec((NC, BM, D), lambda i: (0, i, 0)),
            pl.BlockSpec((D, D), lambda i: (0, 0)),
            pl.BlockSpec((D, D), lambda i: (0, 0)),
            pl.BlockSpec((1, D), lambda i: (0, 0)),
        ],
        out_specs=pl.BlockSpec((BM, D), lambda i: (i, 0)),
        out_shape=jax.ShapeDtypeStruct((N, D), jnp.float32),
    )(x, p, w1, w2, b2d)


def kernel(x, edge_index, W, b):
    src = edge_index[0]
    dst = edge_index[1].reshape(NW, C, K)
    p = _sc_partials(x, src, dst)
    w1 = W[:, :D]
    w2 = W[:, D:]
    return _tc_combine(x, p, w1, w2, b.reshape(1, D))


# speedup vs baseline: 1.0537x; 1.0198x over previous
import functools

import jax
import jax.numpy as jnp
from jax import lax
from jax.experimental import pallas as pl
from jax.experimental.pallas import tpu as pltpu
from jax.experimental.pallas import tpu_sc as plsc

N, E, D = 10000, 320000, 128
NC, NS = 2, 16
NW = NC * NS
EPW = E // NW
K = 80
C = 125
NP = 10240
RPT = NP // NS


def _sc_partials(x, src, dst):
    mesh = plsc.VectorSubcoreMesh(core_axis_name="c", subcore_axis_name="s")

    @functools.partial(
        pl.kernel,
        out_type=jax.ShapeDtypeStruct((NC, NP, D), jnp.float32),
        mesh=mesh,
        scratch_types=[
            pltpu.VMEM((2, K), jnp.int32),
            pltpu.VMEM((C, K), jnp.int32),
            pltpu.VMEM((3, K, D), jnp.float32),
            pltpu.VMEM_SHARED((NP, D), jnp.float32),
            pltpu.SemaphoreType.DMA,
            pltpu.SemaphoreType.DMA,
            pltpu.SemaphoreType.DMA,
        ],
    )
    def sc_kernel(x_hbm, src_hbm, dst_hbm, out_hbm,
                  src_v, dst_v, rows_v, agg_sh, sem_g, sem_i, sem_s):
        cid = lax.axis_index("c")
        sid = lax.axis_index("s")
        wid = sid * NC + cid
        zv = jnp.zeros((16,), jnp.float32)

        def zero_row(i, carry):
            for j in range(8):
                rows_v[0, i, pl.ds(16 * j, 16)] = zv
            return carry

        lax.fori_loop(0, K, zero_row, 0)

        def zero_slice(r, carry):
            pltpu.sync_copy(rows_v.at[0],
                            agg_sh.at[pl.ds(sid * RPT + r * K, K)])
            return carry

        lax.fori_loop(0, RPT // K, zero_slice, 0)
        pltpu.sync_copy(dst_hbm.at[wid], dst_v)
        plsc.subcore_barrier()

        ibase = wid * EPW
        pltpu.sync_copy(src_hbm.at[pl.ds(ibase, K)], src_v.at[0])
        pltpu.async_copy(x_hbm.at[src_v.at[0]], rows_v.at[0], sem_g)
        pltpu.async_copy(src_hbm.at[pl.ds(ibase + K, K)], src_v.at[1], sem_i)

        def body(g, carry):
            @pl.when(g >= 2)
            def _free_buf():
                pltpu.make_async_copy(
                    rows_v.at[(g + 1) % 3],
                    agg_sh.at[dst_v.at[jnp.maximum(g - 2, 0)]], sem_s).wait()

            pltpu.make_async_copy(
                src_hbm.at[pl.ds(ibase + jnp.minimum(g + 1, C - 1) * K, K)],
                src_v.at[(g + 1) % 2], sem_i).wait()
            pltpu.async_copy(x_hbm.at[src_v.at[(g + 1) % 2]],
                             rows_v.at[(g + 1) % 3], sem_g)
            pltpu.make_async_copy(x_hbm.at[src_v.at[g % 2]],
                                  rows_v.at[g % 3], sem_g).wait()
            pltpu.async_copy(
                src_hbm.at[pl.ds(ibase + jnp.minimum(g + 2, C - 1) * K, K)],
                src_v.at[g % 2], sem_i)
            pltpu.async_copy(rows_v.at[g % 3], agg_sh.at[dst_v.at[g]],
                             sem_s, add=True)
            return carry

        lax.fori_loop(0, C, body, 0)
        pltpu.make_async_copy(x_hbm.at[src_v.at[C % 2]],
                              rows_v.at[C % 3], sem_g).wait()
        pltpu.make_async_copy(src_hbm.at[pl.ds(ibase + (C - 1) * K, K)],
                              src_v.at[(C - 1) % 2], sem_i).wait()

        def drain(g, carry):
            pltpu.make_async_copy(rows_v.at[g % 3],
                                  agg_sh.at[dst_v.at[g]], sem_s).wait()
            return carry

        lax.fori_loop(C - 2, C, drain, 0)
        plsc.subcore_barrier()
        pltpu.sync_copy(agg_sh.at[pl.ds(sid * RPT, RPT)],
                        out_hbm.at[cid].at[pl.ds(sid * RPT, RPT)])

    return sc_kernel(x, src, dst)


BM = 2000


def _tc_combine(x, p, w1, w2, b2d):
    def body(x_ref, p_ref, w1_ref, w2_ref, b_ref, o_ref):
        agg = p_ref[0] + p_ref[1]
        dn = (((1,), (1,)), ((), ()))
        o_ref[...] = (
            lax.dot_general(x_ref[...], w1_ref[...], dn,
                            preferred_element_type=jnp.float32)
            + lax.dot_general(agg, w2_ref[...], dn,
                              preferred_element_type=jnp.float32)
            + b_ref[...]
        )

    return pl.pallas_call(
        body,
        grid=(N // BM,),
        in_specs=[
            pl.BlockSpec((BM, D), lambda i: (i, 0)),
            pl.BlockSpec((NC, BM, D), lambda i: (0, i, 0)),
            pl.BlockSpec((D, D), lambda i: (0, 0)),
            pl.BlockSpec((D, D), lambda i: (0, 0)),
            pl.BlockSpec((1, D), lambda i: (0, 0)),
        ],
        out_specs=pl.BlockSpec((BM, D), lambda i: (i, 0)),
        out_shape=jax.ShapeDtypeStruct((N, D), jnp.float32),
    )(x, p, w1, w2, b2d)


def kernel(x, edge_index, W, b):
    src = edge_index[0]
    dst = edge_index[1].reshape(NW, C, K)
    p = _sc_partials(x, src, dst)
    w1 = W[:, :D]
    w2 = W[:, D:]
    return _tc_combine(x, p, w1, w2, b.reshape(1, D))
